# baseline (speedup 1.0000x reference)
import functools

import jax
import jax.numpy as jnp
from jax import lax
from jax.experimental import pallas as pl
from jax.experimental.pallas import tpu as pltpu
from jax.experimental.pallas import tpu_sc as plsc

B = 16384
D = 64
DP = 128
N = 2 * B
NC = 2
NS = 16
EPT = B // NS
CH = 128
NCH = EPT // CH
NROW = B // CH
QR = B // 2
QPT = QR // NS

_MESH = plsc.VectorSubcoreMesh(
    core_axis_name="c", subcore_axis_name="s", num_cores=NC, num_subcores=NS
)

_f32 = jnp.float32
_i32 = jnp.int32


def _rsqrt16(x):
    i = lax.bitcast_convert_type(x, _i32)
    i = jnp.int32(0x5F3759DF) - lax.shift_right_arithmetic(i, 1)
    y = lax.bitcast_convert_type(i, _f32)
    for _ in range(4):
        y = y * (1.5 - 0.5 * x * y * y)
    return y


def _prep_body(ui2, ii2, utab, itab, z0, dinv, idx_deg, idx_emb, ones_v,
               deg_v, dinv_v, rows_c, deg_sh, sem_a, sem_b):
    c = lax.axis_index("c")
    s = lax.axis_index("s")
    gbase = c * B + s * EPT

    for i in range(CH // 16):
        ones_v[pl.ds(i * 16, 16)] = jnp.full((16,), 1.0, _f32)

    def fill_body(i, carry):
        deg_v[pl.ds(i * 16, 16)] = jnp.full((16,), 1.0, _f32)
        return carry

    lax.fori_loop(0, EPT // 16, fill_body, 0)
    pltpu.sync_copy(deg_v, deg_sh.at[pl.ds(s * EPT, EPT)])

    @pl.when(c == 0)
    def _():
        pltpu.sync_copy(ii2.at[pl.ds(s * NCH, NCH)], idx_deg)
        pltpu.sync_copy(ui2.at[pl.ds(s * NCH, NCH)], idx_emb)

    @pl.when(c != 0)
    def _():
        pltpu.sync_copy(ui2.at[pl.ds(s * NCH, NCH)], idx_deg)
        pltpu.sync_copy(ii2.at[pl.ds(s * NCH, NCH)], idx_emb)

    def _emb_gather(j, buf, dsem):
        @pl.when(c == 0)
        def _():
            pltpu.async_copy(utab.at[idx_emb.at[j]], buf, dsem)

        @pl.when(c != 0)
        def _():
            pltpu.async_copy(itab.at[idx_emb.at[j]], buf, dsem)

    bufs = (rows_c.at[0], rows_c.at[1])
    sems = (sem_a, sem_b)
    _emb_gather(0, bufs[0], sems[0])

    plsc.subcore_barrier()
    for j in range(NCH):
        pltpu.sync_copy(ones_v, deg_sh.at[idx_deg.at[j]], add=True)
    plsc.subcore_barrier()

    pltpu.sync_copy(deg_sh.at[pl.ds(s * EPT, EPT)], deg_v)

    def newton_body(i, carry):
        x = deg_v[pl.ds(i * 16, 16)]
        dinv_v[pl.ds(i * 16, 16)] = _rsqrt16(x)
        return carry

    lax.fori_loop(0, EPT // 16, newton_body, 0)
    pltpu.sync_copy(dinv_v, dinv.at[pl.ds(gbase, EPT)])

    for j in range(NCH):
        b = j % 2
        if j + 1 < NCH:
            _emb_gather(j + 1, bufs[1 - b], sems[1 - b])
        pltpu.make_async_copy(utab.at[idx_emb.at[j]], bufs[b], sems[b]).wait()

        def scale_body(m, carry):
            dv = dinv_v[pl.ds(j * CH + m * 16, 16)]
            for t in range(16):
                dsplat = jnp.broadcast_to(dv[t], (16,))
                r = m * 16 + t
                for k in range(D // 16):
                    rows_c[b, r, pl.ds(k * 16, 16)] = (
                        rows_c[b, r, pl.ds(k * 16, 16)] * dsplat
                    )
            return carry

        lax.fori_loop(0, CH // 16, scale_body, 0)
        pltpu.sync_copy(bufs[b], z0.at[pl.ds(gbase + j * CH, CH)])


def _agg_body(z, uo2, ui2, ii2, out, idx_src, ldst, scat, rows_c, out_q,
              g0, g1, g2, g3, s0, s1, s2, s3, isem):
    gsem = (g0, g1, g2, g3)
    ssem = (s0, s1, s2, s3)
    c = lax.axis_index("c")
    s = lax.axis_index("s")

    @pl.when(c == 0)
    def _():
        pltpu.sync_copy(uo2.at[pl.ds(s * NCH, NCH)], idx_src)
        pltpu.sync_copy(ii2.at[pl.ds(s * NCH, NCH)], ldst)

    @pl.when(c != 0)
    def _():
        pltpu.sync_copy(ii2.at[pl.ds(s * NCH, NCH)], idx_src)
        pltpu.sync_copy(ui2.at[pl.ds(s * NCH, NCH)], ldst)

    NB = 2
    bufs = tuple(rows_c.at[k] for k in range(NB))
    for p in range(2):
        qbase = p * QR
        init_src = z.at[pl.ds(c * B + qbase + s * QPT, QPT)]
        init_dst = out_q.at[pl.ds(s * QPT, QPT)]
        pltpu.async_copy(init_src, init_dst, s0)
        for j in range(NCH):

            def selq_body(i, carry):
                v = ldst[j, pl.ds(i * 16, 16)] - qbase
                m = (v >= 0) & (v < QR)
                scat[j, pl.ds(i * 16, 16)] = jnp.where(m, v, QR)
                return carry

            lax.fori_loop(0, CH // 16, selq_body, 0)
        pltpu.make_async_copy(init_src, init_dst, s0).wait()
        plsc.subcore_barrier()
        for k in range(NB - 1):
            pltpu.async_copy(z.at[idx_src.at[k]], bufs[k], gsem[k])
        for j in range(NCH):
            b = j % NB
            nxt = j + NB - 1
            if nxt < NCH:
                pltpu.async_copy(
                    z.at[idx_src.at[nxt]], bufs[nxt % NB], gsem[nxt % NB]
                )
            pltpu.make_async_copy(
                z.at[idx_src.at[j]], bufs[b], gsem[b]
            ).wait()
            pltpu.sync_copy(bufs[b], out_q.at[scat.at[j]], add=True)
        plsc.subcore_barrier()
        pltpu.sync_copy(
            out_q.at[pl.ds(s * QPT, QPT)],
            out.at[pl.ds(c * B + qbase + s * QPT, QPT)],
        )


_prep = pl.kernel(
    _prep_body,
    out_type=(
        jax.ShapeDtypeStruct((N, DP), _f32),
        jax.ShapeDtypeStruct((N,), _f32),
    ),
    mesh=_MESH,
    scratch_types=[
        pltpu.VMEM((NCH, CH), _i32),
        pltpu.VMEM((NCH, CH), _i32),
        pltpu.VMEM((CH,), _f32),
        pltpu.VMEM((EPT,), _f32),
        pltpu.VMEM((EPT,), _f32),
        pltpu.VMEM((2, CH, DP), _f32),
        pltpu.VMEM_SHARED((B,), _f32),
        pltpu.SemaphoreType.DMA,
        pltpu.SemaphoreType.DMA,
    ],
)

_agg = pl.kernel(
    _agg_body,
    out_type=jax.ShapeDtypeStruct((N, DP), _f32),
    mesh=_MESH,
    scratch_types=[
        pltpu.VMEM((NCH, CH), _i32),
        pltpu.VMEM((NCH, CH), _i32),
        pltpu.VMEM((NCH, CH), _i32),
        pltpu.VMEM((2, CH, DP), _f32),
        pltpu.VMEM_SHARED((QR + 1, DP), _f32),
        pltpu.SemaphoreType.DMA,
        pltpu.SemaphoreType.DMA,
        pltpu.SemaphoreType.DMA,
        pltpu.SemaphoreType.DMA,
        pltpu.SemaphoreType.DMA,
        pltpu.SemaphoreType.DMA,
        pltpu.SemaphoreType.DMA,
        pltpu.SemaphoreType.DMA,
        pltpu.SemaphoreType.DMA,
    ],
)

_BLK = 2048


def _padk_body(t_ref, o_ref):
    o_ref[:, :D] = t_ref[...]


_padk = pl.pallas_call(
    _padk_body,
    grid=(B // _BLK,),
    in_specs=[pl.BlockSpec((_BLK, D), lambda i: (i, 0))],
    out_specs=pl.BlockSpec((_BLK, DP), lambda i: (i, 0)),
    out_shape=jax.ShapeDtypeStruct((B, DP), _f32),
)


def _tc_body(last, s_ref, d_ref, w_ref, b_ref, o_ref):
    a = s_ref[:, :D] * d_ref[...]
    h = jnp.dot(a, w_ref[...], preferred_element_type=_f32) + b_ref[...]
    h = jnp.maximum(h, 0.0)
    if last:
        o_ref[...] = h
    else:
        o_ref[...] = jnp.concatenate([h * d_ref[...], jnp.zeros_like(h)], axis=1)


def _tc_layer(sagg, dinv2, W, b2, last):
    if last:
        halves = []
        for h in range(2):
            off = h * (B // _BLK)
            halves.append(pl.pallas_call(
                functools.partial(_tc_body, True),
                grid=(B // _BLK,),
                in_specs=[
                    pl.BlockSpec((_BLK, DP), lambda i, off=off: (i + off, 0)),
                    pl.BlockSpec((_BLK, 1), lambda i, off=off: (i + off, 0)),
                    pl.BlockSpec((D, D), lambda i: (0, 0)),
                    pl.BlockSpec((1, D), lambda i: (0, 0)),
                ],
                out_specs=pl.BlockSpec((_BLK, D), lambda i: (i, 0)),
                out_shape=jax.ShapeDtypeStruct((B, D), _f32),
            )(sagg, dinv2, W, b2))
        return tuple(halves)
    return pl.pallas_call(
        functools.partial(_tc_body, False),
        grid=(N // _BLK,),
        in_specs=[
            pl.BlockSpec((_BLK, DP), lambda i: (i, 0)),
            pl.BlockSpec((_BLK, 1), lambda i: (i, 0)),
            pl.BlockSpec((D, D), lambda i: (0, 0)),
            pl.BlockSpec((1, D), lambda i: (0, 0)),
        ],
        out_specs=pl.BlockSpec((_BLK, DP), lambda i: (i, 0)),
        out_shape=jax.ShapeDtypeStruct((N, DP), _f32),
    )(sagg, dinv2, W, b2)


def kernel(user_indices, item_indices, user_table, item_table,
           W0, b0, W1, b1, W2, b2):
    ui = user_indices.astype(_i32)
    ii = item_indices.astype(_i32)
    ui2 = ui.reshape(NROW, CH)
    ii2 = ii.reshape(NROW, CH)
    uo2 = (ui + B).reshape(NROW, CH)
    ut_p = _padk(user_table)
    it_p = _padk(item_table)

    z0, dinv = _prep(ui2, ii2, ut_p, it_p)
    dinv2 = dinv.reshape(N, 1)

    x = z0
    for (W, b, last) in ((W0, b0, False), (W1, b1, False), (W2, b2, True)):
        sagg = _agg(x, uo2, ui2, ii2)
        x = _tc_layer(sagg, dinv2, W, b.reshape(1, D), last)

    return x


# speedup vs baseline: 1.2849x; 1.2849x over previous
import functools

import jax
import jax.numpy as jnp
from jax import lax
from jax.experimental import pallas as pl
from jax.experimental.pallas import tpu as pltpu
from jax.experimental.pallas import tpu_sc as plsc

B = 16384
D = 64
DP = 128
N = 2 * B
NC = 2
NS = 16
EPT = B // NS
CH = 128
NCH = EPT // CH
NROW = B // CH
QR = B // 2
QPT = QR // NS

_MESH = plsc.VectorSubcoreMesh(
    core_axis_name="c", subcore_axis_name="s", num_cores=NC, num_subcores=NS
)

_f32 = jnp.float32
_i32 = jnp.int32


def _rsqrt16(x):
    i = lax.bitcast_convert_type(x, _i32)
    i = jnp.int32(0x5F3759DF) - lax.shift_right_arithmetic(i, 1)
    y = lax.bitcast_convert_type(i, _f32)
    for _ in range(4):
        y = y * (1.5 - 0.5 * x * y * y)
    return y


def _prep_body(ui2, ii2, utab, itab, z0, dinv, idx_deg, idx_emb, ones_v,
               deg_v, dinv_v, rows_c, deg_sh, sem_a, sem_b):
    c = lax.axis_index("c")
    s = lax.axis_index("s")
    gbase = c * B + s * EPT

    for i in range(CH // 16):
        ones_v[pl.ds(i * 16, 16)] = jnp.full((16,), 1.0, _f32)

    def fill_body(i, carry):
        deg_v[pl.ds(i * 16, 16)] = jnp.full((16,), 1.0, _f32)
        return carry

    lax.fori_loop(0, EPT // 16, fill_body, 0)
    pltpu.sync_copy(deg_v, deg_sh.at[pl.ds(s * EPT, EPT)])

    @pl.when(c == 0)
    def _():
        pltpu.sync_copy(ii2.at[pl.ds(s * NCH, NCH)], idx_deg)
        pltpu.sync_copy(ui2.at[pl.ds(s * NCH, NCH)], idx_emb)

    @pl.when(c != 0)
    def _():
        pltpu.sync_copy(ui2.at[pl.ds(s * NCH, NCH)], idx_deg)
        pltpu.sync_copy(ii2.at[pl.ds(s * NCH, NCH)], idx_emb)

    def _emb_gather(j, buf, dsem):
        @pl.when(c == 0)
        def _():
            pltpu.async_copy(utab.at[idx_emb.at[j]], buf, dsem)

        @pl.when(c != 0)
        def _():
            pltpu.async_copy(itab.at[idx_emb.at[j]], buf, dsem)

    bufs = (rows_c.at[0], rows_c.at[1])
    sems = (sem_a, sem_b)
    _emb_gather(0, bufs[0], sems[0])

    plsc.subcore_barrier()
    for j in range(NCH):
        pltpu.sync_copy(ones_v, deg_sh.at[idx_deg.at[j]], add=True)
    plsc.subcore_barrier()

    pltpu.sync_copy(deg_sh.at[pl.ds(s * EPT, EPT)], deg_v)

    def newton_body(i, carry):
        x = deg_v[pl.ds(i * 16, 16)]
        dinv_v[pl.ds(i * 16, 16)] = _rsqrt16(x)
        return carry

    lax.fori_loop(0, EPT // 16, newton_body, 0)
    pltpu.sync_copy(dinv_v, dinv.at[pl.ds(gbase, EPT)])

    for j in range(NCH):
        b = j % 2
        if j + 1 < NCH:
            _emb_gather(j + 1, bufs[1 - b], sems[1 - b])
        pltpu.make_async_copy(utab.at[idx_emb.at[j]], bufs[b], sems[b]).wait()

        def scale_body(m, carry):
            dv = dinv_v[pl.ds(j * CH + m * 16, 16)]
            for t in range(16):
                dsplat = jnp.broadcast_to(dv[t], (16,))
                r = m * 16 + t
                for k in range(D // 16):
                    rows_c[b, r, pl.ds(k * 16, 16)] = (
                        rows_c[b, r, pl.ds(k * 16, 16)] * dsplat
                    )
            return carry

        lax.fori_loop(0, CH // 16, scale_body, 0)
        pltpu.sync_copy(bufs[b], z0.at[pl.ds(gbase + j * CH, CH)])


def _agg_body(z, uo2, ui2, ii2, out, idx_src, ldst, scat, rows_c, out_q,
              g0, g1, g2, g3, s0, s1, s2, s3, isem):
    gsem = (g0, g1, g2, g3)
    ssem = (s0, s1, s2, s3)
    c = lax.axis_index("c")
    s = lax.axis_index("s")

    @pl.when(c == 0)
    def _():
        pltpu.sync_copy(uo2.at[pl.ds(s * NCH, NCH)], idx_src)
        pltpu.sync_copy(ii2.at[pl.ds(s * NCH, NCH)], ldst)

    @pl.when(c != 0)
    def _():
        pltpu.sync_copy(ii2.at[pl.ds(s * NCH, NCH)], idx_src)
        pltpu.sync_copy(ui2.at[pl.ds(s * NCH, NCH)], ldst)

    NB = 2
    bufs = tuple(rows_c.at[k] for k in range(NB))
    for p in range(2):
        qbase = p * QR
        init_src = z.at[pl.ds(c * B + qbase + s * QPT, QPT)]
        init_dst = out_q.at[pl.ds(s * QPT, QPT)]
        pltpu.async_copy(init_src, init_dst, s0)
        for j in range(NCH):

            def selq_body(i, carry):
                v = ldst[j, pl.ds(i * 16, 16)] - qbase
                m = (v >= 0) & (v < QR)
                scat[j, pl.ds(i * 16, 16)] = jnp.where(m, v, QR)
                return carry

            lax.fori_loop(0, CH // 16, selq_body, 0)
        pltpu.make_async_copy(init_src, init_dst, s0).wait()
        plsc.subcore_barrier()
        for k in range(NB - 1):
            pltpu.async_copy(z.at[idx_src.at[k]], bufs[k], gsem[k])
        for j in range(NCH):
            b = j % NB
            nxt = j + NB - 1
            if nxt < NCH:
                pltpu.async_copy(
                    z.at[idx_src.at[nxt]], bufs[nxt % NB], gsem[nxt % NB]
                )
            pltpu.make_async_copy(
                z.at[idx_src.at[j]], bufs[b], gsem[b]
            ).wait()
            pltpu.sync_copy(bufs[b], out_q.at[scat.at[j]], add=True)
        plsc.subcore_barrier()
        pltpu.sync_copy(
            out_q.at[pl.ds(s * QPT, QPT)],
            out.at[pl.ds(c * B + qbase + s * QPT, QPT)],
        )


_prep = pl.kernel(
    _prep_body,
    out_type=(
        jax.ShapeDtypeStruct((N, DP), _f32),
        jax.ShapeDtypeStruct((N,), _f32),
    ),
    mesh=_MESH,
    scratch_types=[
        pltpu.VMEM((NCH, CH), _i32),
        pltpu.VMEM((NCH, CH), _i32),
        pltpu.VMEM((CH,), _f32),
        pltpu.VMEM((EPT,), _f32),
        pltpu.VMEM((EPT,), _f32),
        pltpu.VMEM((2, CH, DP), _f32),
        pltpu.VMEM_SHARED((B,), _f32),
        pltpu.SemaphoreType.DMA,
        pltpu.SemaphoreType.DMA,
    ],
)

_agg = pl.kernel(
    _agg_body,
    out_type=jax.ShapeDtypeStruct((N, DP), _f32),
    mesh=_MESH,
    scratch_types=[
        pltpu.VMEM((NCH, CH), _i32),
        pltpu.VMEM((NCH, CH), _i32),
        pltpu.VMEM((NCH, CH), _i32),
        pltpu.VMEM((2, CH, DP), _f32),
        pltpu.VMEM_SHARED((QR + 1, DP), _f32),
        pltpu.SemaphoreType.DMA,
        pltpu.SemaphoreType.DMA,
        pltpu.SemaphoreType.DMA,
        pltpu.SemaphoreType.DMA,
        pltpu.SemaphoreType.DMA,
        pltpu.SemaphoreType.DMA,
        pltpu.SemaphoreType.DMA,
        pltpu.SemaphoreType.DMA,
        pltpu.SemaphoreType.DMA,
    ],
)

_BLK = 4096


def _tc_body(last, s_ref, d_ref, w_ref, b_ref, o_ref):
    a = s_ref[:, :D] * d_ref[...]
    h = jnp.dot(a, w_ref[...], preferred_element_type=_f32) + b_ref[...]
    h = jnp.maximum(h, 0.0)
    if last:
        o_ref[...] = h
    else:
        o_ref[...] = jnp.concatenate([h * d_ref[...], jnp.zeros_like(h)], axis=1)


def _tc_layer(sagg, dinv2, W, b2, last):
    if last:
        halves = []
        for h in range(2):
            off = h * (B // _BLK)
            halves.append(pl.pallas_call(
                functools.partial(_tc_body, True),
                grid=(B // _BLK,),
                in_specs=[
                    pl.BlockSpec((_BLK, DP), lambda i, off=off: (i + off, 0)),
                    pl.BlockSpec((_BLK, 1), lambda i, off=off: (i + off, 0)),
                    pl.BlockSpec((D, D), lambda i: (0, 0)),
                    pl.BlockSpec((1, D), lambda i: (0, 0)),
                ],
                out_specs=pl.BlockSpec((_BLK, D), lambda i: (i, 0)),
                out_shape=jax.ShapeDtypeStruct((B, D), _f32),
            )(sagg, dinv2, W, b2))
        return tuple(halves)
    return pl.pallas_call(
        functools.partial(_tc_body, False),
        grid=(N // _BLK,),
        in_specs=[
            pl.BlockSpec((_BLK, DP), lambda i: (i, 0)),
            pl.BlockSpec((_BLK, 1), lambda i: (i, 0)),
            pl.BlockSpec((D, D), lambda i: (0, 0)),
            pl.BlockSpec((1, D), lambda i: (0, 0)),
        ],
        out_specs=pl.BlockSpec((_BLK, DP), lambda i: (i, 0)),
        out_shape=jax.ShapeDtypeStruct((N, DP), _f32),
    )(sagg, dinv2, W, b2)


def kernel(user_indices, item_indices, user_table, item_table,
           W0, b0, W1, b1, W2, b2):
    ui = user_indices.astype(_i32)
    ii = item_indices.astype(_i32)
    ui2 = ui.reshape(NROW, CH)
    ii2 = ii.reshape(NROW, CH)
    uo2 = (ui + B).reshape(NROW, CH)
    ut_p = jnp.pad(user_table[:B], ((0, 0), (0, DP - D)))
    it_p = jnp.pad(item_table[:B], ((0, 0), (0, DP - D)))

    z0, dinv = _prep(ui2, ii2, ut_p, it_p)
    dinv2 = dinv.reshape(N, 1)

    x = z0
    for (W, b, last) in ((W0, b0, False), (W1, b1, False), (W2, b2, True)):
        sagg = _agg(x, uo2, ui2, ii2)
        x = _tc_layer(sagg, dinv2, W, b.reshape(1, D), last)

    return x


# speedup vs baseline: 1.3177x; 1.0255x over previous
import functools

import jax
import jax.numpy as jnp
from jax import lax
from jax.experimental import pallas as pl
from jax.experimental.pallas import tpu as pltpu
from jax.experimental.pallas import tpu_sc as plsc

B = 16384
D = 64
DP = 128
N = 2 * B
NC = 2
NS = 16
EPT = B // NS
CH = 128
NCH = EPT // CH
NROW = B // CH
QR = B // 2
QPT = QR // NS

_MESH = plsc.VectorSubcoreMesh(
    core_axis_name="c", subcore_axis_name="s", num_cores=NC, num_subcores=NS
)

_f32 = jnp.float32
_i32 = jnp.int32


def _rsqrt16(x):
    i = lax.bitcast_convert_type(x, _i32)
    i = jnp.int32(0x5F3759DF) - lax.shift_right_arithmetic(i, 1)
    y = lax.bitcast_convert_type(i, _f32)
    for _ in range(4):
        y = y * (1.5 - 0.5 * x * y * y)
    return y


def _prep_body(ui2, ii2, utab, itab, z0, dinv, idx_deg, idx_emb, ones_v,
               deg_v, dinv_v, rows_c, deg_sh, sem_a, sem_b):
    c = lax.axis_index("c")
    s = lax.axis_index("s")
    gbase = c * B + s * EPT

    for i in range(CH // 16):
        ones_v[pl.ds(i * 16, 16)] = jnp.full((16,), 1.0, _f32)

    def fill_body(i, carry):
        deg_v[pl.ds(i * 16, 16)] = jnp.full((16,), 1.0, _f32)
        return carry

    lax.fori_loop(0, EPT // 16, fill_body, 0)
    pltpu.sync_copy(deg_v, deg_sh.at[pl.ds(s * EPT, EPT)])

    @pl.when(c == 0)
    def _():
        pltpu.sync_copy(ii2.at[pl.ds(s * NCH, NCH)], idx_deg)
        pltpu.sync_copy(ui2.at[pl.ds(s * NCH, NCH)], idx_emb)

    @pl.when(c != 0)
    def _():
        pltpu.sync_copy(ui2.at[pl.ds(s * NCH, NCH)], idx_deg)
        pltpu.sync_copy(ii2.at[pl.ds(s * NCH, NCH)], idx_emb)

    def _emb_gather(j, buf, dsem):
        @pl.when(c == 0)
        def _():
            pltpu.async_copy(utab.at[idx_emb.at[j]], buf, dsem)

        @pl.when(c != 0)
        def _():
            pltpu.async_copy(itab.at[idx_emb.at[j]], buf, dsem)

    bufs = (rows_c.at[0], rows_c.at[1])
    sems = (sem_a, sem_b)
    _emb_gather(0, bufs[0], sems[0])

    plsc.subcore_barrier()
    for j in range(NCH):
        pltpu.sync_copy(ones_v, deg_sh.at[idx_deg.at[j]], add=True)
    plsc.subcore_barrier()

    pltpu.sync_copy(deg_sh.at[pl.ds(s * EPT, EPT)], deg_v)

    def newton_body(i, carry):
        x = deg_v[pl.ds(i * 16, 16)]
        dinv_v[pl.ds(i * 16, 16)] = _rsqrt16(x)
        return carry

    lax.fori_loop(0, EPT // 16, newton_body, 0)
    pltpu.sync_copy(dinv_v, dinv.at[pl.ds(gbase, EPT)])

    for j in range(NCH):
        b = j % 2
        if j + 1 < NCH:
            _emb_gather(j + 1, bufs[1 - b], sems[1 - b])
        pltpu.make_async_copy(utab.at[idx_emb.at[j]], bufs[b], sems[b]).wait()

        def scale_body(m, carry):
            dv = dinv_v[pl.ds(j * CH + m * 16, 16)]
            for t in range(16):
                dsplat = jnp.broadcast_to(dv[t], (16,))
                r = m * 16 + t
                for k in range(D // 16):
                    rows_c[b, r, pl.ds(k * 16, 16)] = (
                        rows_c[b, r, pl.ds(k * 16, 16)] * dsplat
                    )
            return carry

        lax.fori_loop(0, CH // 16, scale_body, 0)
        pltpu.sync_copy(bufs[b], z0.at[pl.ds(gbase + j * CH, CH)])


def _agg_body(z, uo2, ui2, ii2, out, idx_src, ldst, scat, rows_c, out_q,
              g0, g1, g2, g3, s0, s1, s2, s3, isem):
    gsem = (g0, g1, g2, g3)
    ssem = (s0, s1, s2, s3)
    c = lax.axis_index("c")
    s = lax.axis_index("s")

    @pl.when(c == 0)
    def _():
        pltpu.sync_copy(uo2.at[pl.ds(s * NCH, NCH)], idx_src)
        pltpu.sync_copy(ii2.at[pl.ds(s * NCH, NCH)], ldst)

    @pl.when(c != 0)
    def _():
        pltpu.sync_copy(ii2.at[pl.ds(s * NCH, NCH)], idx_src)
        pltpu.sync_copy(ui2.at[pl.ds(s * NCH, NCH)], ldst)

    NB = 2
    bufs = tuple(rows_c.at[k] for k in range(NB))
    for p in range(2):
        qbase = p * QR
        init_src = z.at[pl.ds(c * B + qbase + s * QPT, QPT)]
        init_dst = out_q.at[pl.ds(s * QPT, QPT)]
        pltpu.async_copy(init_src, init_dst, s0)
        for j in range(NCH):

            def selq_body(i, carry):
                v = ldst[j, pl.ds(i * 16, 16)] - qbase
                m = (v >= 0) & (v < QR)
                scat[j, pl.ds(i * 16, 16)] = jnp.where(m, v, QR)
                return carry

            lax.fori_loop(0, CH // 16, selq_body, 0)
        pltpu.make_async_copy(init_src, init_dst, s0).wait()
        plsc.subcore_barrier()
        for k in range(NB - 1):
            pltpu.async_copy(z.at[idx_src.at[k]], bufs[k], gsem[k])
        for j in range(NCH):
            b = j % NB
            nxt = j + NB - 1
            if nxt < NCH:
                pltpu.async_copy(
                    z.at[idx_src.at[nxt]], bufs[nxt % NB], gsem[nxt % NB]
                )
            pltpu.make_async_copy(
                z.at[idx_src.at[j]], bufs[b], gsem[b]
            ).wait()
            pltpu.sync_copy(bufs[b], out_q.at[scat.at[j]], add=True)
        plsc.subcore_barrier()
        pltpu.sync_copy(
            out_q.at[pl.ds(s * QPT, QPT)],
            out.at[pl.ds(c * B + qbase + s * QPT, QPT)],
        )


_prep = pl.kernel(
    _prep_body,
    out_type=(
        jax.ShapeDtypeStruct((N, DP), _f32),
        jax.ShapeDtypeStruct((N,), _f32),
    ),
    mesh=_MESH,
    scratch_types=[
        pltpu.VMEM((NCH, CH), _i32),
        pltpu.VMEM((NCH, CH), _i32),
        pltpu.VMEM((CH,), _f32),
        pltpu.VMEM((EPT,), _f32),
        pltpu.VMEM((EPT,), _f32),
        pltpu.VMEM((2, CH, DP), _f32),
        pltpu.VMEM_SHARED((B,), _f32),
        pltpu.SemaphoreType.DMA,
        pltpu.SemaphoreType.DMA,
    ],
)

_agg = pl.kernel(
    _agg_body,
    out_type=jax.ShapeDtypeStruct((N, DP), _f32),
    mesh=_MESH,
    scratch_types=[
        pltpu.VMEM((NCH, CH), _i32),
        pltpu.VMEM((NCH, CH), _i32),
        pltpu.VMEM((NCH, CH), _i32),
        pltpu.VMEM((2, CH, DP), _f32),
        pltpu.VMEM_SHARED((QR + 1, DP), _f32),
        pltpu.SemaphoreType.DMA,
        pltpu.SemaphoreType.DMA,
        pltpu.SemaphoreType.DMA,
        pltpu.SemaphoreType.DMA,
        pltpu.SemaphoreType.DMA,
        pltpu.SemaphoreType.DMA,
        pltpu.SemaphoreType.DMA,
        pltpu.SemaphoreType.DMA,
        pltpu.SemaphoreType.DMA,
    ],
)

_BLK = 8192


def _tc_body(last, s_ref, d_ref, w_ref, b_ref, o_ref):
    a = s_ref[:, :D] * d_ref[...]
    h = jnp.dot(a, w_ref[...], preferred_element_type=_f32) + b_ref[...]
    h = jnp.maximum(h, 0.0)
    if last:
        o_ref[...] = h
    else:
        o_ref[...] = jnp.concatenate([h * d_ref[...], jnp.zeros_like(h)], axis=1)


def _tc_layer(sagg, dinv2, W, b2, last):
    if last:
        halves = []
        for h in range(2):
            off = h * (B // _BLK)
            halves.append(pl.pallas_call(
                functools.partial(_tc_body, True),
                grid=(B // _BLK,),
                in_specs=[
                    pl.BlockSpec((_BLK, DP), lambda i, off=off: (i + off, 0)),
                    pl.BlockSpec((_BLK, 1), lambda i, off=off: (i + off, 0)),
                    pl.BlockSpec((D, D), lambda i: (0, 0)),
                    pl.BlockSpec((1, D), lambda i: (0, 0)),
                ],
                out_specs=pl.BlockSpec((_BLK, D), lambda i: (i, 0)),
                out_shape=jax.ShapeDtypeStruct((B, D), _f32),
            )(sagg, dinv2, W, b2))
        return tuple(halves)
    return pl.pallas_call(
        functools.partial(_tc_body, False),
        grid=(N // _BLK,),
        in_specs=[
            pl.BlockSpec((_BLK, DP), lambda i: (i, 0)),
            pl.BlockSpec((_BLK, 1), lambda i: (i, 0)),
            pl.BlockSpec((D, D), lambda i: (0, 0)),
            pl.BlockSpec((1, D), lambda i: (0, 0)),
        ],
        out_specs=pl.BlockSpec((_BLK, DP), lambda i: (i, 0)),
        out_shape=jax.ShapeDtypeStruct((N, DP), _f32),
    )(sagg, dinv2, W, b2)


def kernel(user_indices, item_indices, user_table, item_table,
           W0, b0, W1, b1, W2, b2):
    ui = user_indices.astype(_i32)
    ii = item_indices.astype(_i32)
    ui2 = ui.reshape(NROW, CH)
    ii2 = ii.reshape(NROW, CH)
    uo2 = (ui + B).reshape(NROW, CH)
    ut_p = jnp.pad(user_table[:B], ((0, 0), (0, DP - D)))
    it_p = jnp.pad(item_table[:B], ((0, 0), (0, DP - D)))

    z0, dinv = _prep(ui2, ii2, ut_p, it_p)
    dinv2 = dinv.reshape(N, 1)

    x = z0
    for (W, b, last) in ((W0, b0, False), (W1, b1, False), (W2, b2, True)):
        sagg = _agg(x, uo2, ui2, ii2)
        x = _tc_layer(sagg, dinv2, W, b.reshape(1, D), last)

    return x
